# initial kernel scaffold (unmeasured)
import jax
import jax.numpy as jnp
from jax import lax
from jax.experimental import pallas as pl
from jax.experimental.pallas import tpu as pltpu

N_DEV = 4
S = 1024
D = 2048
DC = 128
H = 16
DH = 128
DR = 32
SCALE = (DH + DR) ** -0.5


def kernel(x, Wdkv, Wuk, Wuv, Wq, Wqr, Wkr, Wo):
    x2 = x.reshape(S, D)

    def body(x_ref, wdkv_ref, wuk_ref, wuv_ref, wq_ref, wqr_ref, wkr_ref,
             wo_ref, out_ref,
             c_loc, comm_c, comm_uk, comm_uv, k_acc, v_acc, qo,
             send_sems, recv_sems):
        my = lax.axis_index("i")
        left = (my + N_DEV - 1) % N_DEV
        right = (my + 1) % N_DEV

        barrier_sem = pltpu.get_barrier_semaphore()
        for nbr in (left, right):
            pl.semaphore_signal(barrier_sem, inc=1, device_id=(nbr,),
                                device_id_type=pl.DeviceIdType.MESH)
        pl.semaphore_wait(barrier_sem, 2)

        xv = x_ref[...]

        c = jnp.dot(xv, wdkv_ref[...], preferred_element_type=jnp.float32)
        c_loc[...] = c
        k_acc[...] = jnp.dot(c, wuk_ref[...], preferred_element_type=jnp.float32)
        v_acc[...] = jnp.dot(c, wuv_ref[...], preferred_element_type=jnp.float32)

        for h in range(N_DEV - 1):
            srcs = (
                (c_loc if h == 0 else comm_c.at[h - 1], comm_c),
                (wuk_ref if h == 0 else comm_uk.at[h - 1], comm_uk),
                (wuv_ref if h == 0 else comm_uv.at[h - 1], comm_uv),
            )
            rdmas = []
            for t, (src, dstbuf) in enumerate(srcs):
                r = pltpu.make_async_remote_copy(
                    src_ref=src,
                    dst_ref=dstbuf.at[h],
                    send_sem=send_sems.at[h, t],
                    recv_sem=recv_sems.at[h, t],
                    device_id=(right,),
                    device_id_type=pl.DeviceIdType.MESH,
                )
                r.start()
                rdmas.append(r)
            for r in rdmas:
                r.wait()
            k_acc[...] += jnp.dot(comm_c[h], comm_uk[h],
                                  preferred_element_type=jnp.float32)
            v_acc[...] += jnp.dot(comm_c[h], comm_uv[h],
                                  preferred_element_type=jnp.float32)

        qo[...] = jnp.dot(xv, wq_ref[...], preferred_element_type=jnp.float32)
        qr = jnp.dot(xv, wqr_ref[...], preferred_element_type=jnp.float32)
        kr = jnp.dot(xv, wkr_ref[...], preferred_element_type=jnp.float32)

        for hd in range(H):
            q = qo[:, hd * DH:(hd + 1) * DH]
            k = k_acc[:, hd * DH:(hd + 1) * DH]
            s = lax.dot_general(q, k, (((1,), (1,)), ((), ())),
                                preferred_element_type=jnp.float32)
            s += lax.dot_general(qr[:, hd * DR:(hd + 1) * DR], kr,
                                 (((1,), (1,)), ((), ())),
                                 preferred_element_type=jnp.float32)
            s *= SCALE
            m = jnp.max(s, axis=1, keepdims=True)
            p = jnp.exp(s - m)
            p = p / jnp.sum(p, axis=1, keepdims=True)
            o = jnp.dot(p, v_acc[:, hd * DH:(hd + 1) * DH],
                        preferred_element_type=jnp.float32)
            qo[:, hd * DH:(hd + 1) * DH] = o

        out_ref[...] = jnp.dot(qo[...], wo_ref[...],
                               preferred_element_type=jnp.float32)

    out = pl.pallas_call(
        body,
        out_shape=jax.ShapeDtypeStruct((S, D), jnp.float32),
        in_specs=[pl.BlockSpec(memory_space=pltpu.VMEM)] * 8,
        out_specs=pl.BlockSpec(memory_space=pltpu.VMEM),
        scratch_shapes=[
            pltpu.VMEM((S, DC), jnp.float32),
            pltpu.VMEM((N_DEV - 1, S, DC), jnp.float32),
            pltpu.VMEM((N_DEV - 1, DC, D), jnp.float32),
            pltpu.VMEM((N_DEV - 1, DC, D), jnp.float32),
            pltpu.VMEM((S, D), jnp.float32),
            pltpu.VMEM((S, D), jnp.float32),
            pltpu.VMEM((S, D), jnp.float32),
            pltpu.SemaphoreType.DMA((N_DEV - 1, 3)),
            pltpu.SemaphoreType.DMA((N_DEV - 1, 3)),
        ],
        compiler_params=pltpu.CompilerParams(collective_id=0),
    )(x2, Wdkv, Wuk, Wuv, Wq, Wqr, Wkr, Wo)
    return out.reshape(1, S, D)


# baseline (device time: 261554 ns/iter reference)
import jax
import jax.numpy as jnp
from jax import lax
from jax.experimental import pallas as pl
from jax.experimental.pallas import tpu as pltpu

N_DEV = 4
S = 1024
D = 2048
DC = 128
H = 16
DH = 128
DR = 32
SCALE = (DH + DR) ** -0.5
F32 = jnp.float32


def _gather_body(x_ref, wdkv_ref, wuk_ref, wuv_ref, wqr_ref, wkr_ref,
                 c_out, uk_out, uv_out, qr_out, kr_out,
                 send_sems, recv_sems):
    my = lax.axis_index("i")
    left = (my + N_DEV - 1) % N_DEV
    right = (my + 1) % N_DEV

    barrier_sem = pltpu.get_barrier_semaphore()
    for nbr in (left, right):
        pl.semaphore_signal(barrier_sem, inc=1, device_id=(nbr,),
                            device_id_type=pl.DeviceIdType.MESH)
    pl.semaphore_wait(barrier_sem, 2)

    xv = x_ref[...]
    kr_out[...] = jnp.dot(xv, wkr_ref[...], preferred_element_type=F32)
    wqr_v = wqr_ref[...]
    for hd in range(H):
        qr_out[hd] = jnp.dot(xv, wqr_v[:, hd * DR:(hd + 1) * DR],
                             preferred_element_type=F32)
    c_out[0] = jnp.dot(xv, wdkv_ref[...], preferred_element_type=F32)
    uk_out[0] = wuk_ref[...]
    uv_out[0] = wuv_ref[...]

    for h in range(N_DEV - 1):
        rdmas = []
        for t, buf in enumerate((c_out, uk_out, uv_out)):
            r = pltpu.make_async_remote_copy(
                src_ref=buf.at[h],
                dst_ref=buf.at[h + 1],
                send_sem=send_sems.at[h, t],
                recv_sem=recv_sems.at[h, t],
                device_id=(right,),
                device_id_type=pl.DeviceIdType.MESH,
            )
            r.start()
            rdmas.append(r)
        for r in rdmas:
            r.wait()


def _attn_body(x_ref, c_ref, uk_ref, uv_ref, wq_ref, qr_ref, kr_ref,
               wo_ref, out_ref):
    hd = pl.program_id(0)
    xv = x_ref[...]

    k_h = jnp.dot(c_ref[0], uk_ref[0], preferred_element_type=F32)
    v_h = jnp.dot(c_ref[0], uv_ref[0], preferred_element_type=F32)
    for k in range(1, N_DEV):
        k_h += jnp.dot(c_ref[k], uk_ref[k], preferred_element_type=F32)
        v_h += jnp.dot(c_ref[k], uv_ref[k], preferred_element_type=F32)

    q_h = jnp.dot(xv, wq_ref[...], preferred_element_type=F32)
    qr_h = qr_ref[0]

    s = lax.dot_general(q_h, k_h, (((1,), (1,)), ((), ())),
                        preferred_element_type=F32)
    s += lax.dot_general(qr_h, kr_ref[...], (((1,), (1,)), ((), ())),
                         preferred_element_type=F32)
    s *= SCALE
    m = jnp.max(s, axis=1, keepdims=True)
    p = jnp.exp(s - m)
    p = p / jnp.sum(p, axis=1, keepdims=True)
    o_h = jnp.dot(p, v_h, preferred_element_type=F32)
    contrib = jnp.dot(o_h, wo_ref[...], preferred_element_type=F32)

    @pl.when(hd == 0)
    def _():
        out_ref[...] = contrib

    @pl.when(hd > 0)
    def _():
        out_ref[...] += contrib


def kernel(x, Wdkv, Wuk, Wuv, Wq, Wqr, Wkr, Wo):
    x2 = x.reshape(S, D)

    c_sl, uk_sl, uv_sl, qr_t, kr = pl.pallas_call(
        _gather_body,
        out_shape=(
            jax.ShapeDtypeStruct((N_DEV, S, DC), F32),
            jax.ShapeDtypeStruct((N_DEV, DC, D), F32),
            jax.ShapeDtypeStruct((N_DEV, DC, D), F32),
            jax.ShapeDtypeStruct((H, S, DR), F32),
            jax.ShapeDtypeStruct((S, DR), F32),
        ),
        in_specs=[pl.BlockSpec(memory_space=pltpu.VMEM)] * 6,
        out_specs=(pl.BlockSpec(memory_space=pltpu.VMEM),) * 5,
        scratch_shapes=[
            pltpu.SemaphoreType.DMA((N_DEV - 1, 3)),
            pltpu.SemaphoreType.DMA((N_DEV - 1, 3)),
        ],
        compiler_params=pltpu.CompilerParams(collective_id=0),
    )(x2, Wdkv, Wuk, Wuv, Wqr, Wkr)

    out = pl.pallas_call(
        _attn_body,
        grid=(H,),
        out_shape=jax.ShapeDtypeStruct((S, D), F32),
        in_specs=[
            pl.BlockSpec((S, D), lambda h: (0, 0)),
            pl.BlockSpec((N_DEV, S, DC), lambda h: (0, 0, 0)),
            pl.BlockSpec((N_DEV, DC, DH), lambda h: (0, 0, h)),
            pl.BlockSpec((N_DEV, DC, DH), lambda h: (0, 0, h)),
            pl.BlockSpec((D, DH), lambda h: (0, h)),
            pl.BlockSpec((1, S, DR), lambda h: (h, 0, 0)),
            pl.BlockSpec((S, DR), lambda h: (0, 0)),
            pl.BlockSpec((DH, D), lambda h: (h, 0)),
        ],
        out_specs=pl.BlockSpec((S, D), lambda h: (0, 0)),
        compiler_params=pltpu.CompilerParams(
            dimension_semantics=("arbitrary",),
        ),
    )(x2, c_sl, uk_sl, uv_sl, Wq, qr_t, kr, Wo)
    return out.reshape(1, S, D)


# device time: 113128 ns/iter; 2.3120x vs baseline; 2.3120x over previous
import jax
import jax.numpy as jnp
from jax import lax
from jax.experimental import pallas as pl
from jax.experimental.pallas import tpu as pltpu

N_DEV = 4
S = 1024
D = 2048
DC = 128
DC_ALL = N_DEV * DC
H = 16
HL = H // N_DEV
DH = 128
HB = HL * DH
DR = 32
SCALE = (DH + DR) ** -0.5
F32 = jnp.float32


def _peer_barrier(peers):
    barrier_sem = pltpu.get_barrier_semaphore()
    for nbr in peers:
        pl.semaphore_signal(barrier_sem, inc=1, device_id=(nbr,),
                            device_id_type=pl.DeviceIdType.MESH)
    pl.semaphore_wait(barrier_sem, len(peers))


def _gather_body(x_ref, wdkv_ref, wuk_ref, wuv_ref, wqb_ref, wqrb_ref,
                 wkr_ref,
                 c_out, uk_out, uv_out, q_out, qr_out, kr_out,
                 send_sems, recv_sems):
    my = lax.axis_index("i")
    _peer_barrier([(my + j) % N_DEV for j in range(1, N_DEV)])

    sends = []
    for p_rel in range(1, N_DEV):
        p = (my + p_rel) % N_DEV
        for t, (w_ref, dst) in enumerate(((wuk_ref, uk_out),
                                          (wuv_ref, uv_out))):
            r = pltpu.make_async_remote_copy(
                src_ref=w_ref.at[:, pl.ds(p * HB, HB)],
                dst_ref=dst.at[pl.ds(my * DC, DC), :],
                send_sem=send_sems.at[p_rel - 1, t],
                recv_sem=recv_sems.at[3 - p_rel, t],
                device_id=(p,),
                device_id_type=pl.DeviceIdType.MESH,
            )
            r.start()
            sends.append(r)

    xv = x_ref[...]
    c = jnp.dot(xv, wdkv_ref[...], preferred_element_type=F32)
    c_out[:, pl.ds(my * DC, DC)] = c
    for p_rel in range(1, N_DEV):
        p = (my + p_rel) % N_DEV
        r = pltpu.make_async_remote_copy(
            src_ref=c_out.at[:, pl.ds(my * DC, DC)],
            dst_ref=c_out.at[:, pl.ds(my * DC, DC)],
            send_sem=send_sems.at[p_rel - 1, 2],
            recv_sem=recv_sems.at[3 - p_rel, 2],
            device_id=(p,),
            device_id_type=pl.DeviceIdType.MESH,
        )
        r.start()
        sends.append(r)

    uk_out[pl.ds(my * DC, DC), :] = wuk_ref[:, pl.ds(my * HB, HB)]
    uv_out[pl.ds(my * DC, DC), :] = wuv_ref[:, pl.ds(my * HB, HB)]
    kr_out[...] = jnp.dot(xv, wkr_ref[...], preferred_element_type=F32)
    q_out[...] = jnp.dot(xv, wqb_ref[...], preferred_element_type=F32)
    qr_out[...] = jnp.dot(xv, wqrb_ref[...], preferred_element_type=F32)

    for r_slot in range(N_DEV - 1):
        o = (my + r_slot + 1) % N_DEV
        for t, dst in enumerate((uk_out, uv_out)):
            rcv = pltpu.make_async_remote_copy(
                src_ref=dst.at[pl.ds(o * DC, DC), :],
                dst_ref=dst.at[pl.ds(o * DC, DC), :],
                send_sem=send_sems.at[r_slot, t],
                recv_sem=recv_sems.at[r_slot, t],
                device_id=(my,),
                device_id_type=pl.DeviceIdType.MESH,
            )
            rcv.wait_recv()
        rcv = pltpu.make_async_remote_copy(
            src_ref=c_out.at[:, pl.ds(o * DC, DC)],
            dst_ref=c_out.at[:, pl.ds(o * DC, DC)],
            send_sem=send_sems.at[r_slot, 2],
            recv_sem=recv_sems.at[r_slot, 2],
            device_id=(my,),
            device_id_type=pl.DeviceIdType.MESH,
        )
        rcv.wait_recv()
    for s in sends:
        s.wait_send()


def _attn_body(c_ref, uk_ref, uv_ref, q_ref, qr_ref, kr_ref, o_out):
    cv = c_ref[...]
    krv = kr_ref[...]
    for h in range(HL):
        k_h = jnp.dot(cv, uk_ref[:, h * DH:(h + 1) * DH],
                      preferred_element_type=F32)
        v_h = jnp.dot(cv, uv_ref[:, h * DH:(h + 1) * DH],
                      preferred_element_type=F32)
        q_h = q_ref[:, h * DH:(h + 1) * DH]
        qr_h = qr_ref[:, h * DR:(h + 1) * DR]
        s = lax.dot_general(q_h, k_h, (((1,), (1,)), ((), ())),
                            preferred_element_type=F32)
        s += lax.dot_general(qr_h, krv, (((1,), (1,)), ((), ())),
                             preferred_element_type=F32)
        s *= SCALE
        m = jnp.max(s, axis=1, keepdims=True)
        p = jnp.exp(s - m)
        p = p / jnp.sum(p, axis=1, keepdims=True)
        o_out[:, h * DH:(h + 1) * DH] = jnp.dot(
            p, v_h, preferred_element_type=F32)


def _ogather_body(o_ref, of_out, send_sems, recv_sems):
    my = lax.axis_index("i")
    _peer_barrier([(my + j) % N_DEV for j in range(1, N_DEV)])

    of_out[:, pl.ds(my * HB, HB)] = o_ref[...]
    sends = []
    for p_rel in range(1, N_DEV):
        p = (my + p_rel) % N_DEV
        r = pltpu.make_async_remote_copy(
            src_ref=of_out.at[:, pl.ds(my * HB, HB)],
            dst_ref=of_out.at[:, pl.ds(my * HB, HB)],
            send_sem=send_sems.at[p_rel - 1],
            recv_sem=recv_sems.at[3 - p_rel],
            device_id=(p,),
            device_id_type=pl.DeviceIdType.MESH,
        )
        r.start()
        sends.append(r)
    for r_slot in range(N_DEV - 1):
        o = (my + r_slot + 1) % N_DEV
        rcv = pltpu.make_async_remote_copy(
            src_ref=of_out.at[:, pl.ds(o * HB, HB)],
            dst_ref=of_out.at[:, pl.ds(o * HB, HB)],
            send_sem=send_sems.at[r_slot],
            recv_sem=recv_sems.at[r_slot],
            device_id=(my,),
            device_id_type=pl.DeviceIdType.MESH,
        )
        rcv.wait_recv()
    for s in sends:
        s.wait_send()


def _proj_body(of_ref, wo_ref, out_ref):
    out_ref[...] = jnp.dot(of_ref[...], wo_ref[...],
                           preferred_element_type=F32)


def kernel(x, Wdkv, Wuk, Wuv, Wq, Wqr, Wkr, Wo):
    x2 = x.reshape(S, D)
    my = lax.axis_index("i")
    wq_blk = lax.dynamic_slice(Wq, (0, my * HB), (D, HB))
    wqr_blk = lax.dynamic_slice(Wqr, (0, my * HL * DR), (D, HL * DR))

    c_full, uk_c, uv_c, q_my, qr_my, kr = pl.pallas_call(
        _gather_body,
        out_shape=(
            jax.ShapeDtypeStruct((S, DC_ALL), F32),
            jax.ShapeDtypeStruct((DC_ALL, HB), F32),
            jax.ShapeDtypeStruct((DC_ALL, HB), F32),
            jax.ShapeDtypeStruct((S, HB), F32),
            jax.ShapeDtypeStruct((S, HL * DR), F32),
            jax.ShapeDtypeStruct((S, DR), F32),
        ),
        in_specs=[pl.BlockSpec(memory_space=pltpu.VMEM)] * 7,
        out_specs=(pl.BlockSpec(memory_space=pltpu.VMEM),) * 6,
        scratch_shapes=[
            pltpu.SemaphoreType.DMA((N_DEV - 1, 3)),
            pltpu.SemaphoreType.DMA((N_DEV - 1, 3)),
        ],
        compiler_params=pltpu.CompilerParams(collective_id=0),
    )(x2, Wdkv, Wuk, Wuv, wq_blk, wqr_blk, Wkr)

    o_local = pl.pallas_call(
        _attn_body,
        out_shape=jax.ShapeDtypeStruct((S, HB), F32),
        in_specs=[pl.BlockSpec(memory_space=pltpu.VMEM)] * 6,
        out_specs=pl.BlockSpec(memory_space=pltpu.VMEM),
    )(c_full, uk_c, uv_c, q_my, qr_my, kr)

    o_full = pl.pallas_call(
        _ogather_body,
        out_shape=jax.ShapeDtypeStruct((S, D), F32),
        in_specs=[pl.BlockSpec(memory_space=pltpu.VMEM)],
        out_specs=pl.BlockSpec(memory_space=pltpu.VMEM),
        scratch_shapes=[
            pltpu.SemaphoreType.DMA((N_DEV - 1,)),
            pltpu.SemaphoreType.DMA((N_DEV - 1,)),
        ],
        compiler_params=pltpu.CompilerParams(collective_id=1),
    )(o_local)

    out = pl.pallas_call(
        _proj_body,
        grid=(N_DEV,),
        out_shape=jax.ShapeDtypeStruct((S, D), F32),
        in_specs=[
            pl.BlockSpec((S, D), lambda j: (0, 0)),
            pl.BlockSpec((D, HB), lambda j: (0, j)),
        ],
        out_specs=pl.BlockSpec((S, HB), lambda j: (0, j)),
        compiler_params=pltpu.CompilerParams(
            dimension_semantics=("arbitrary",),
        ),
    )(o_full, Wo)
    return out.reshape(1, S, D)


# device time: 83470 ns/iter; 3.1335x vs baseline; 1.3553x over previous
import jax
import jax.numpy as jnp
from jax import lax
from jax.experimental import pallas as pl
from jax.experimental.pallas import tpu as pltpu

N_DEV = 4
S = 1024
D = 2048
DC = 128
DC_ALL = N_DEV * DC
H = 16
HL = H // N_DEV
DH = 128
HB = HL * DH
DR = 32
SCALE = (DH + DR) ** -0.5
F32 = jnp.float32
BF16 = jnp.bfloat16


def _peer_barrier(peers):
    barrier_sem = pltpu.get_barrier_semaphore()
    for nbr in peers:
        pl.semaphore_signal(barrier_sem, inc=1, device_id=(nbr,),
                            device_id_type=pl.DeviceIdType.MESH)
    pl.semaphore_wait(barrier_sem, len(peers))


def _gather_body(x_ref, wdkv_ref, wuk_ref, wuv_ref, wqb_ref, wqrb_ref,
                 wkr_ref,
                 c_out, uk_out, uv_out, q_out, qr_out, kr_out,
                 uk_b, uv_b, send_sems, recv_sems):
    my = lax.axis_index("i")
    _peer_barrier([(my + j) % N_DEV for j in range(1, N_DEV)])

    uk_b[...] = wuk_ref[...].astype(BF16)
    uv_b[...] = wuv_ref[...].astype(BF16)
    sends = []
    for p_rel in range(1, N_DEV):
        p = (my + p_rel) % N_DEV
        for t, (w_b, dst) in enumerate(((uk_b, uk_out), (uv_b, uv_out))):
            r = pltpu.make_async_remote_copy(
                src_ref=w_b.at[:, pl.ds(p * HB, HB)],
                dst_ref=dst.at[pl.ds(my * DC, DC), :],
                send_sem=send_sems.at[p_rel - 1, t],
                recv_sem=recv_sems.at[3 - p_rel, t],
                device_id=(p,),
                device_id_type=pl.DeviceIdType.MESH,
            )
            r.start()
            sends.append(r)

    xv = x_ref[...]
    c = jnp.dot(xv, wdkv_ref[...], preferred_element_type=F32)
    c_out[:, pl.ds(my * DC, DC)] = c.astype(BF16)
    for p_rel in range(1, N_DEV):
        p = (my + p_rel) % N_DEV
        r = pltpu.make_async_remote_copy(
            src_ref=c_out.at[:, pl.ds(my * DC, DC)],
            dst_ref=c_out.at[:, pl.ds(my * DC, DC)],
            send_sem=send_sems.at[p_rel - 1, 2],
            recv_sem=recv_sems.at[3 - p_rel, 2],
            device_id=(p,),
            device_id_type=pl.DeviceIdType.MESH,
        )
        r.start()
        sends.append(r)

    uk_out[pl.ds(my * DC, DC), :] = uk_b[:, pl.ds(my * HB, HB)]
    uv_out[pl.ds(my * DC, DC), :] = uv_b[:, pl.ds(my * HB, HB)]
    kr_out[...] = jnp.dot(xv, wkr_ref[...], preferred_element_type=F32)
    q_out[...] = jnp.dot(xv, wqb_ref[...], preferred_element_type=F32)
    qr_out[...] = jnp.dot(xv, wqrb_ref[...], preferred_element_type=F32)

    for r_slot in range(N_DEV - 1):
        o = (my + r_slot + 1) % N_DEV
        for t, dst in enumerate((uk_out, uv_out)):
            rcv = pltpu.make_async_remote_copy(
                src_ref=dst.at[pl.ds(o * DC, DC), :],
                dst_ref=dst.at[pl.ds(o * DC, DC), :],
                send_sem=send_sems.at[r_slot, t],
                recv_sem=recv_sems.at[r_slot, t],
                device_id=(my,),
                device_id_type=pl.DeviceIdType.MESH,
            )
            rcv.wait_recv()
        rcv = pltpu.make_async_remote_copy(
            src_ref=c_out.at[:, pl.ds(o * DC, DC)],
            dst_ref=c_out.at[:, pl.ds(o * DC, DC)],
            send_sem=send_sems.at[r_slot, 2],
            recv_sem=recv_sems.at[r_slot, 2],
            device_id=(my,),
            device_id_type=pl.DeviceIdType.MESH,
        )
        rcv.wait_recv()
    for s in sends:
        s.wait_send()


def _attn_body(c_ref, uk_ref, uv_ref, q_ref, qr_ref, kr_ref, o_out):
    cv = c_ref[...]
    krv = kr_ref[...]
    for h in range(HL):
        k_h = jnp.dot(cv, uk_ref[:, h * DH:(h + 1) * DH],
                      preferred_element_type=F32)
        v_h = jnp.dot(cv, uv_ref[:, h * DH:(h + 1) * DH],
                      preferred_element_type=F32)
        q_h = q_ref[:, h * DH:(h + 1) * DH]
        qr_h = qr_ref[:, h * DR:(h + 1) * DR]
        s = lax.dot_general(q_h, k_h, (((1,), (1,)), ((), ())),
                            preferred_element_type=F32)
        s += lax.dot_general(qr_h, krv, (((1,), (1,)), ((), ())),
                             preferred_element_type=F32)
        s *= SCALE
        m = jnp.max(s, axis=1, keepdims=True)
        p = jnp.exp(s - m)
        p = p / jnp.sum(p, axis=1, keepdims=True)
        o_out[:, h * DH:(h + 1) * DH] = jnp.dot(
            p, v_h, preferred_element_type=F32).astype(BF16)


def _proj_body(o_ref, wo_ref, out_ref, o_recv, send_sems, recv_sems):
    my = lax.axis_index("i")
    _peer_barrier([(my + j) % N_DEV for j in range(1, N_DEV)])

    sends = []
    for p_rel in range(1, N_DEV):
        p = (my + p_rel) % N_DEV
        r = pltpu.make_async_remote_copy(
            src_ref=o_ref,
            dst_ref=o_recv.at[3 - p_rel],
            send_sem=send_sems.at[p_rel - 1],
            recv_sem=recv_sems.at[3 - p_rel],
            device_id=(p,),
            device_id_type=pl.DeviceIdType.MESH,
        )
        r.start()
        sends.append(r)

    out_ref[...] = jnp.dot(o_ref[...], wo_ref[pl.ds(my * HB, HB), :],
                           preferred_element_type=F32)
    for r_slot in range(N_DEV - 1):
        o = (my + r_slot + 1) % N_DEV
        rcv = pltpu.make_async_remote_copy(
            src_ref=o_recv.at[r_slot],
            dst_ref=o_recv.at[r_slot],
            send_sem=send_sems.at[r_slot],
            recv_sem=recv_sems.at[r_slot],
            device_id=(my,),
            device_id_type=pl.DeviceIdType.MESH,
        )
        rcv.wait_recv()
        out_ref[...] += jnp.dot(o_recv[r_slot], wo_ref[pl.ds(o * HB, HB), :],
                                preferred_element_type=F32)
    for s in sends:
        s.wait_send()


def kernel(x, Wdkv, Wuk, Wuv, Wq, Wqr, Wkr, Wo):
    x2 = x.reshape(S, D)
    my = lax.axis_index("i")
    wq_blk = lax.dynamic_slice(Wq, (0, my * HB), (D, HB))
    wqr_blk = lax.dynamic_slice(Wqr, (0, my * HL * DR), (D, HL * DR))
    wo_b = Wo.astype(BF16)

    c_full, uk_c, uv_c, q_my, qr_my, kr = pl.pallas_call(
        _gather_body,
        out_shape=(
            jax.ShapeDtypeStruct((S, DC_ALL), BF16),
            jax.ShapeDtypeStruct((DC_ALL, HB), BF16),
            jax.ShapeDtypeStruct((DC_ALL, HB), BF16),
            jax.ShapeDtypeStruct((S, HB), F32),
            jax.ShapeDtypeStruct((S, HL * DR), F32),
            jax.ShapeDtypeStruct((S, DR), F32),
        ),
        in_specs=[pl.BlockSpec(memory_space=pltpu.VMEM)] * 7,
        out_specs=(pl.BlockSpec(memory_space=pltpu.VMEM),) * 6,
        scratch_shapes=[
            pltpu.VMEM((DC, D), BF16),
            pltpu.VMEM((DC, D), BF16),
            pltpu.SemaphoreType.DMA((N_DEV - 1, 3)),
            pltpu.SemaphoreType.DMA((N_DEV - 1, 3)),
        ],
        compiler_params=pltpu.CompilerParams(collective_id=0),
    )(x2, Wdkv, Wuk, Wuv, wq_blk, wqr_blk, Wkr)

    o_local = pl.pallas_call(
        _attn_body,
        out_shape=jax.ShapeDtypeStruct((S, HB), BF16),
        in_specs=[pl.BlockSpec(memory_space=pltpu.VMEM)] * 6,
        out_specs=pl.BlockSpec(memory_space=pltpu.VMEM),
    )(c_full, uk_c, uv_c, q_my, qr_my, kr)

    out = pl.pallas_call(
        _proj_body,
        out_shape=jax.ShapeDtypeStruct((S, D), F32),
        in_specs=[pl.BlockSpec(memory_space=pltpu.VMEM)] * 2,
        out_specs=pl.BlockSpec(memory_space=pltpu.VMEM),
        scratch_shapes=[
            pltpu.VMEM((N_DEV - 1, S, HB), BF16),
            pltpu.SemaphoreType.DMA((N_DEV - 1,)),
            pltpu.SemaphoreType.DMA((N_DEV - 1,)),
        ],
        compiler_params=pltpu.CompilerParams(collective_id=1),
    )(o_local, wo_b)
    return out.reshape(1, S, D)


# device time: 80298 ns/iter; 3.2573x vs baseline; 1.0395x over previous
import jax
import jax.numpy as jnp
from jax import lax
from jax.experimental import pallas as pl
from jax.experimental.pallas import tpu as pltpu

N_DEV = 4
S = 1024
D = 2048
DC = 128
DC_ALL = N_DEV * DC
H = 16
HL = H // N_DEV
DH = 128
HB = HL * DH
DR = 32
SCALE = (DH + DR) ** -0.5
F32 = jnp.float32
BF16 = jnp.bfloat16


def _peer_barrier(peers):
    barrier_sem = pltpu.get_barrier_semaphore()
    for nbr in peers:
        pl.semaphore_signal(barrier_sem, inc=1, device_id=(nbr,),
                            device_id_type=pl.DeviceIdType.MESH)
    pl.semaphore_wait(barrier_sem, len(peers))


def _gather_body(x_ref, wdkv_ref, wuk_ref, wuv_ref, wqb_ref, wqrb_ref,
                 wkr_ref,
                 c_out, uk_out, uv_out, q_out, qr_out, kr_out,
                 uk_b, uv_b, xb, send_sems, recv_sems):
    my = lax.axis_index("i")
    _peer_barrier([(my + j) % N_DEV for j in range(1, N_DEV)])

    uk_b[...] = wuk_ref[...].astype(BF16)
    uv_b[...] = wuv_ref[...].astype(BF16)
    sends = []
    for p_rel in range(1, N_DEV):
        p = (my + p_rel) % N_DEV
        for t, (w_b, dst) in enumerate(((uk_b, uk_out), (uv_b, uv_out))):
            r = pltpu.make_async_remote_copy(
                src_ref=w_b.at[:, pl.ds(p * HB, HB)],
                dst_ref=dst.at[pl.ds(my * DC, DC), :],
                send_sem=send_sems.at[p_rel - 1, t],
                recv_sem=recv_sems.at[3 - p_rel, t],
                device_id=(p,),
                device_id_type=pl.DeviceIdType.MESH,
            )
            r.start()
            sends.append(r)

    xb[...] = x_ref[...].astype(BF16)
    xbv = xb[...]
    c = jnp.dot(xbv, wdkv_ref[...].astype(BF16), preferred_element_type=F32)
    c_out[:, pl.ds(my * DC, DC)] = c.astype(BF16)
    for p_rel in range(1, N_DEV):
        p = (my + p_rel) % N_DEV
        r = pltpu.make_async_remote_copy(
            src_ref=c_out.at[:, pl.ds(my * DC, DC)],
            dst_ref=c_out.at[:, pl.ds(my * DC, DC)],
            send_sem=send_sems.at[p_rel - 1, 2],
            recv_sem=recv_sems.at[3 - p_rel, 2],
            device_id=(p,),
            device_id_type=pl.DeviceIdType.MESH,
        )
        r.start()
        sends.append(r)

    uk_out[pl.ds(my * DC, DC), :] = uk_b[:, pl.ds(my * HB, HB)]
    uv_out[pl.ds(my * DC, DC), :] = uv_b[:, pl.ds(my * HB, HB)]
    kr_out[...] = jnp.dot(xbv, wkr_ref[...].astype(BF16),
                          preferred_element_type=F32).astype(BF16)
    q_out[...] = jnp.dot(xbv, wqb_ref[...],
                         preferred_element_type=F32).astype(BF16)
    qr_out[...] = jnp.dot(xbv, wqrb_ref[...],
                          preferred_element_type=F32).astype(BF16)

    for r_slot in range(N_DEV - 1):
        o = (my + r_slot + 1) % N_DEV
        for t, dst in enumerate((uk_out, uv_out)):
            rcv = pltpu.make_async_remote_copy(
                src_ref=dst.at[pl.ds(o * DC, DC), :],
                dst_ref=dst.at[pl.ds(o * DC, DC), :],
                send_sem=send_sems.at[r_slot, t],
                recv_sem=recv_sems.at[r_slot, t],
                device_id=(my,),
                device_id_type=pl.DeviceIdType.MESH,
            )
            rcv.wait_recv()
        rcv = pltpu.make_async_remote_copy(
            src_ref=c_out.at[:, pl.ds(o * DC, DC)],
            dst_ref=c_out.at[:, pl.ds(o * DC, DC)],
            send_sem=send_sems.at[r_slot, 2],
            recv_sem=recv_sems.at[r_slot, 2],
            device_id=(my,),
            device_id_type=pl.DeviceIdType.MESH,
        )
        rcv.wait_recv()
    for s in sends:
        s.wait_send()


def _attn_body(c_ref, uk_ref, uv_ref, q_ref, qr_ref, kr_ref, o_out):
    cv = c_ref[...]
    krv = kr_ref[...]
    for h in range(HL):
        k_h = jnp.dot(cv, uk_ref[:, h * DH:(h + 1) * DH],
                      preferred_element_type=F32).astype(BF16)
        v_h = jnp.dot(cv, uv_ref[:, h * DH:(h + 1) * DH],
                      preferred_element_type=F32).astype(BF16)
        q_h = q_ref[:, h * DH:(h + 1) * DH]
        qr_h = qr_ref[:, h * DR:(h + 1) * DR]
        s = lax.dot_general(q_h, k_h, (((1,), (1,)), ((), ())),
                            preferred_element_type=F32)
        s += lax.dot_general(qr_h, krv, (((1,), (1,)), ((), ())),
                             preferred_element_type=F32)
        s *= SCALE
        m = jnp.max(s, axis=1, keepdims=True)
        p = jnp.exp(s - m)
        p = (p / jnp.sum(p, axis=1, keepdims=True)).astype(BF16)
        o_out[:, h * DH:(h + 1) * DH] = jnp.dot(
            p, v_h, preferred_element_type=F32).astype(BF16)


def _proj_body(o_ref, wo_ref, out_ref, o_recv, wo_s, wo_b,
               send_sems, recv_sems, dma_sems):
    my = lax.axis_index("i")

    def wo_dma(i, buf):
        idx = (my + i) % N_DEV
        return pltpu.make_async_copy(
            wo_ref.at[pl.ds(idx * HB, HB), :],
            wo_s.at[buf],
            dma_sems.at[buf],
        )

    dma0 = wo_dma(0, 0)
    dma0.start()

    _peer_barrier([(my + j) % N_DEV for j in range(1, N_DEV)])
    sends = []
    for p_rel in range(1, N_DEV):
        p = (my + p_rel) % N_DEV
        r = pltpu.make_async_remote_copy(
            src_ref=o_ref,
            dst_ref=o_recv.at[3 - p_rel],
            send_sem=send_sems.at[p_rel - 1],
            recv_sem=recv_sems.at[3 - p_rel],
            device_id=(p,),
            device_id_type=pl.DeviceIdType.MESH,
        )
        r.start()
        sends.append(r)

    dma0.wait()
    dma1 = wo_dma(1, 1)
    dma1.start()
    wo_b[...] = wo_s[0].astype(BF16)
    out_ref[...] = jnp.dot(o_ref[...], wo_b[...],
                           preferred_element_type=F32)
    dmas = [dma0, dma1]
    for r_slot in range(N_DEV - 1):
        buf = (r_slot + 1) % 2
        dmas[r_slot + 1].wait()
        if r_slot < N_DEV - 2:
            nxt = wo_dma(r_slot + 2, r_slot % 2)
            nxt.start()
            dmas.append(nxt)
        wo_b[...] = wo_s[buf].astype(BF16)
        rcv = pltpu.make_async_remote_copy(
            src_ref=o_recv.at[r_slot],
            dst_ref=o_recv.at[r_slot],
            send_sem=send_sems.at[r_slot],
            recv_sem=recv_sems.at[r_slot],
            device_id=(my,),
            device_id_type=pl.DeviceIdType.MESH,
        )
        rcv.wait_recv()
        out_ref[...] += jnp.dot(o_recv[r_slot], wo_b[...],
                                preferred_element_type=F32)
    for s in sends:
        s.wait_send()


def kernel(x, Wdkv, Wuk, Wuv, Wq, Wqr, Wkr, Wo):
    x2 = x.reshape(S, D)
    my = lax.axis_index("i")
    wq_blk = lax.dynamic_slice(Wq, (0, my * HB), (D, HB)).astype(BF16)
    wqr_blk = lax.dynamic_slice(
        Wqr, (0, my * HL * DR), (D, HL * DR)).astype(BF16)

    c_full, uk_c, uv_c, q_my, qr_my, kr = pl.pallas_call(
        _gather_body,
        out_shape=(
            jax.ShapeDtypeStruct((S, DC_ALL), BF16),
            jax.ShapeDtypeStruct((DC_ALL, HB), BF16),
            jax.ShapeDtypeStruct((DC_ALL, HB), BF16),
            jax.ShapeDtypeStruct((S, HB), BF16),
            jax.ShapeDtypeStruct((S, HL * DR), BF16),
            jax.ShapeDtypeStruct((S, DR), BF16),
        ),
        in_specs=[pl.BlockSpec(memory_space=pltpu.VMEM)] * 7,
        out_specs=(pl.BlockSpec(memory_space=pltpu.VMEM),) * 6,
        scratch_shapes=[
            pltpu.VMEM((DC, D), BF16),
            pltpu.VMEM((DC, D), BF16),
            pltpu.VMEM((S, D), BF16),
            pltpu.SemaphoreType.DMA((N_DEV - 1, 3)),
            pltpu.SemaphoreType.DMA((N_DEV - 1, 3)),
        ],
        compiler_params=pltpu.CompilerParams(collective_id=0),
    )(x2, Wdkv, Wuk, Wuv, wq_blk, wqr_blk, Wkr)

    o_local = pl.pallas_call(
        _attn_body,
        out_shape=jax.ShapeDtypeStruct((S, HB), BF16),
        in_specs=[pl.BlockSpec(memory_space=pltpu.VMEM)] * 6,
        out_specs=pl.BlockSpec(memory_space=pltpu.VMEM),
    )(c_full, uk_c, uv_c, q_my, qr_my, kr)

    out = pl.pallas_call(
        _proj_body,
        out_shape=jax.ShapeDtypeStruct((S, D), F32),
        in_specs=[
            pl.BlockSpec(memory_space=pltpu.VMEM),
            pl.BlockSpec(memory_space=pl.ANY),
        ],
        out_specs=pl.BlockSpec(memory_space=pltpu.VMEM),
        scratch_shapes=[
            pltpu.VMEM((N_DEV - 1, S, HB), BF16),
            pltpu.VMEM((2, HB, D), F32),
            pltpu.VMEM((HB, D), BF16),
            pltpu.SemaphoreType.DMA((N_DEV - 1,)),
            pltpu.SemaphoreType.DMA((N_DEV - 1,)),
            pltpu.SemaphoreType.DMA((2,)),
        ],
        compiler_params=pltpu.CompilerParams(collective_id=1),
    )(o_local, Wo)
    return out.reshape(1, S, D)
